# baseline (speedup 1.0000x reference)
import jax
import jax.numpy as jnp
from jax import lax
from jax.experimental import pallas as pl

M = 100000
Z = 128
K = 100
BM = 2048
MPAD = 100352
PADV = 3.0e38


def _dist_body(q_ref, t_ref, q2_ref, t2_ref, out_ref):
    q = q_ref[...]
    t = t_ref[...]
    mm = jax.lax.dot_general(q, t, (((1,), (1,)), ((), ())),
                             preferred_element_type=jnp.float32)
    out_ref[...] = q2_ref[...] + t2_ref[...] - 2.0 * mm


def _distances(q, tpad, q2, t2pad, bt):
    return pl.pallas_call(
        _dist_body,
        grid=(MPAD // BM,),
        in_specs=[
            pl.BlockSpec((bt, Z), lambda i: (0, 0)),
            pl.BlockSpec((BM, Z), lambda i: (i, 0)),
            pl.BlockSpec((bt, 1), lambda i: (0, 0)),
            pl.BlockSpec((1, BM), lambda i: (0, i)),
        ],
        out_specs=pl.BlockSpec((bt, BM), lambda i: (0, i)),
        out_shape=jax.ShapeDtypeStruct((bt, MPAD), jnp.float32),
    )(q, tpad, q2, t2pad)


def kernel(query_batch, targets):
    B, T, Zd = query_batch.shape
    bt = B * T
    q = query_batch.reshape(bt, Zd)
    tpad = jnp.pad(targets, ((0, MPAD - M), (0, 0)))
    q2 = jnp.sum(q * q, axis=1, keepdims=True)
    t2 = jnp.sum(targets * targets, axis=1)
    t2pad = jnp.pad(t2, (0, MPAD - M), constant_values=PADV)[None, :]
    d = _distances(q, tpad, q2, t2pad, bt)
    neg_topk, idx = lax.top_k(-d, K)
    dists = -neg_topk
    probabilities = jax.nn.softmax(-dists, axis=-1)
    states = jnp.take(targets, idx, axis=0)
    probabilities = probabilities.reshape(B, T, K)
    states = states.reshape(B, T, K, Zd)
    return probabilities, states
